# Initial kernel scaffold; baseline (speedup 1.0000x reference)
import jax
import jax.numpy as jnp
from jax.experimental import pallas as pl


def kernel(data, segment_ids):
    raise NotImplementedError("write your pallas kernel here")


# speedup vs baseline: 4.6068x; 4.6068x over previous
import functools

import jax
import jax.numpy as jnp
from jax import lax
from jax.experimental import pallas as pl
from jax.experimental.pallas import tpu as pltpu
from jax.experimental.pallas import tpu_sc as plsc

_N = 320000
_D = 128
_S = 1024
_NC = 2
_NS = 16
_NW = _NC * _NS
_RPW = _N // _NW
_BLK = 128
_NBLK = _RPW // _BLK
_TAIL = _RPW - _NBLK * _BLK
_SROWS = _S // _NS


def _make_sc_segment_sum():
    mesh = plsc.VectorSubcoreMesh(core_axis_name="c", subcore_axis_name="s")

    @functools.partial(
        pl.kernel,
        out_type=jax.ShapeDtypeStruct((_NC, _S, _D), jnp.float32),
        mesh=mesh,
        scratch_types=[
            pltpu.VMEM((_BLK,), jnp.int32),
            pltpu.VMEM((_BLK, _D), jnp.float32),
            pltpu.VMEM((_TAIL,), jnp.int32),
            pltpu.VMEM((_TAIL, _D), jnp.float32),
            pltpu.VMEM((_SROWS, _D), jnp.float32),
            pltpu.VMEM_SHARED((_S, _D), jnp.float32),
        ],
    )
    def seg_sum(data_hbm, seg_hbm, out_hbm, idx_v, rows_v, idxt_v, rowst_v,
                z_v, acc_sh):
        c = lax.axis_index("c")
        s = lax.axis_index("s")
        wid = c * _NS + s

        zero = jnp.zeros((16,), jnp.float32)

        def zrow(i, carry):
            for j in range(_D // 16):
                z_v[i, pl.ds(j * 16, 16)] = zero
            return carry

        lax.fori_loop(0, _SROWS, zrow, 0)
        pltpu.sync_copy(z_v, acc_sh.at[pl.ds(s * _SROWS, _SROWS)])
        plsc.subcore_barrier()

        row0 = wid * _RPW

        def body(i, carry):
            base = row0 + i * _BLK
            pltpu.sync_copy(seg_hbm.at[pl.ds(base, _BLK)], idx_v)
            pltpu.sync_copy(data_hbm.at[pl.ds(base, _BLK)], rows_v)
            pltpu.sync_copy(rows_v, acc_sh.at[idx_v], add=True)
            return carry

        lax.fori_loop(0, _NBLK, body, 0)

        tb = row0 + _NBLK * _BLK
        pltpu.sync_copy(seg_hbm.at[pl.ds(tb, _TAIL)], idxt_v)
        pltpu.sync_copy(data_hbm.at[pl.ds(tb, _TAIL)], rowst_v)
        pltpu.sync_copy(rowst_v, acc_sh.at[idxt_v], add=True)

        plsc.subcore_barrier()

        pltpu.sync_copy(acc_sh.at[pl.ds(s * _SROWS, _SROWS)],
                        out_hbm.at[c, pl.ds(s * _SROWS, _SROWS)])

    return seg_sum


_sc_segment_sum = _make_sc_segment_sum()


def _combine(p_ref, o_ref):
    o_ref[...] = p_ref[0] + p_ref[1]


@jax.jit
def kernel(data, segment_ids):
    seg32 = segment_ids.astype(jnp.int32)
    partials = _sc_segment_sum(data, seg32)
    return pl.pallas_call(
        _combine,
        out_shape=jax.ShapeDtypeStruct((_S, _D), jnp.float32),
    )(partials)


# speedup vs baseline: 7.3386x; 1.5930x over previous
import functools

import jax
import jax.numpy as jnp
from jax import lax
from jax.experimental import pallas as pl
from jax.experimental.pallas import tpu as pltpu
from jax.experimental.pallas import tpu_sc as plsc

_N = 320000
_D = 128
_S = 1024
_NC = 2
_NS = 16
_NW = _NC * _NS
_B = 128
_NBLK_TOT = _N // _B
_BPW = _NBLK_TOT // _NW
_NEXTRA = _NBLK_TOT - _BPW * _NW
_NSLOT = 4
_SROWS = _S // _NS


def _make_sc_segment_sum():
    mesh = plsc.VectorSubcoreMesh(core_axis_name="c", subcore_axis_name="s")

    @functools.partial(
        pl.kernel,
        out_type=jax.ShapeDtypeStruct((_NC, _S, _D), jnp.float32),
        mesh=mesh,
        scratch_types=[
            pltpu.VMEM((_NSLOT, _B), jnp.int32),
            pltpu.VMEM((_NSLOT, _B, _D), jnp.float32),
            pltpu.VMEM((_SROWS, _D), jnp.float32),
            pltpu.VMEM_SHARED((_S, _D), jnp.float32),
            pltpu.SemaphoreType.DMA,
            pltpu.SemaphoreType.DMA,
            pltpu.SemaphoreType.DMA,
            pltpu.SemaphoreType.DMA,
            pltpu.SemaphoreType.DMA,
            pltpu.SemaphoreType.DMA,
            pltpu.SemaphoreType.DMA,
            pltpu.SemaphoreType.DMA,
        ],
    )
    def seg_sum(data_hbm, seg_hbm, out_hbm, ids_v, rows_v, z_v, acc_sh,
                *sems):
        ld_sems = sems[:_NSLOT]
        sc_sems = sems[_NSLOT:]
        c = lax.axis_index("c")
        s = lax.axis_index("s")
        wid = c * _NS + s
        blk0 = wid * _BPW

        zero = jnp.zeros((16,), jnp.float32)

        def zrow(i, carry):
            for j in range(_D // 16):
                z_v[i, pl.ds(j * 16, 16)] = zero
            return carry

        lax.fori_loop(0, _SROWS, zrow, 0)
        pltpu.sync_copy(z_v, acc_sh.at[pl.ds(s * _SROWS, _SROWS)])
        plsc.subcore_barrier()

        pend_ld = {}
        pend_sc = {}

        def start_load(i):
            slot = i % _NSLOT
            base = (blk0 + i) * _B
            a = pltpu.async_copy(seg_hbm.at[pl.ds(base, _B)],
                                 ids_v.at[slot], ld_sems[slot])
            b = pltpu.async_copy(data_hbm.at[pl.ds(base, _B)],
                                 rows_v.at[slot], ld_sems[slot])
            pend_ld[slot] = (a, b)

        for i in range(min(_NSLOT - 1, _BPW)):
            start_load(i)

        for i in range(_BPW):
            slot = i % _NSLOT
            a, b = pend_ld.pop(slot)
            a.wait()
            b.wait()
            pend_sc[slot] = pltpu.async_copy(
                rows_v.at[slot], acc_sh.at[ids_v.at[slot]], sc_sems[slot],
                add=True)
            j = i + _NSLOT - 1
            if j < _BPW:
                jslot = j % _NSLOT
                if jslot in pend_sc:
                    pend_sc.pop(jslot).wait()
                start_load(j)

        for slot in list(pend_sc):
            pend_sc.pop(slot).wait()

        @pl.when(wid < _NEXTRA)
        def _():
            base = (_NW * _BPW + wid) * _B
            pltpu.sync_copy(seg_hbm.at[pl.ds(base, _B)], ids_v.at[0])
            pltpu.sync_copy(data_hbm.at[pl.ds(base, _B)], rows_v.at[0])
            pltpu.sync_copy(rows_v.at[0], acc_sh.at[ids_v.at[0]], add=True)

        plsc.subcore_barrier()

        pltpu.sync_copy(acc_sh.at[pl.ds(s * _SROWS, _SROWS)],
                        out_hbm.at[c, pl.ds(s * _SROWS, _SROWS)])

    return seg_sum


_sc_segment_sum = _make_sc_segment_sum()


def _combine(p_ref, o_ref):
    o_ref[...] = p_ref[0] + p_ref[1]


@jax.jit
def kernel(data, segment_ids):
    seg32 = segment_ids.astype(jnp.int32)
    partials = _sc_segment_sum(data, seg32)
    return pl.pallas_call(
        _combine,
        out_shape=jax.ShapeDtypeStruct((_S, _D), jnp.float32),
    )(partials)
